# baseline (speedup 1.0000x reference)
import functools

import jax
import jax.numpy as jnp
from jax import lax
from jax.experimental import pallas as pl
from jax.experimental.pallas import tpu as pltpu
from jax.experimental.pallas import tpu_sc as plsc

N = 10000
E = 320000
D_FEAT = 128
D_EDGE = 16

NUM_SC = 2
NUM_TEC = 16
FH = D_EDGE // NUM_SC

NPAD = 10240
PAD_ROW = N + 64
BLK = 128
NBLK = E // BLK
COLS_PER_W = 160
NBLK_PAD = NUM_TEC * COLS_PER_W
TCN = 10
CHUNKS = COLS_PER_W // TCN
NBUF = 2


@functools.cache
def _make_sc_scatter_add():
    mesh = plsc.VectorSubcoreMesh(core_axis_name="c", subcore_axis_name="s")
    return functools.partial(
        pl.kernel,
        mesh=mesh,
        compiler_params=pltpu.CompilerParams(use_tc_tiling_on_sc=False,
                                             needs_layout_passes=False),
        out_type=jax.ShapeDtypeStruct((NUM_SC, NUM_TEC, FH * NPAD),
                                      jnp.float32),
        scratch_types=[
            pltpu.VMEM((NBUF, TCN, BLK), jnp.int32),
            pltpu.VMEM((NBUF, TCN, FH, BLK), jnp.float32),
            pltpu.VMEM((FH * NPAD,), jnp.float32),
            pltpu.SemaphoreType.DMA((NBUF,)),
        ],
    )(_sc_scatter_body)


def _sc_scatter_body(idx_hbm, attr_hbm, out_hbm, idx_v, attr_v, acc2, sem_in):
    cid = lax.axis_index("c")
    sid = lax.axis_index("s")

    zvec = jnp.zeros((16,), jnp.float32)

    def zero_row(i, carry):
        acc2[pl.ds(i * 16, 16)] = zvec
        return carry

    lax.fori_loop(0, FH * NPAD // 16, zero_row, 0)

    cbase = sid * COLS_PER_W

    def start_inputs(t, b):
        c0 = cbase + t * TCN
        c0c = jnp.minimum(c0, NBLK - TCN)
        pltpu.async_copy(idx_hbm.at[pl.ds(c0, TCN)], idx_v.at[b],
                         sem_in.at[b])
        pltpu.async_copy(attr_hbm.at[cid, pl.ds(c0c, TCN)], attr_v.at[b],
                         sem_in.at[b])

    for b in range(NBUF):
        start_inputs(b, b)

    def outer(i, carry):
        for b in range(NBUF):
            t = i * NBUF + b
            pltpu.make_async_copy(idx_hbm.at[pl.ds(0, TCN)], idx_v.at[b],
                                  sem_in.at[b]).wait()
            pltpu.make_async_copy(attr_hbm.at[0, pl.ds(0, TCN)],
                                  attr_v.at[b], sem_in.at[b]).wait()
            for tc in range(TCN):
                for j in range(BLK // 16):
                    iv = idx_v[b, tc, pl.ds(j * 16, 16)]
                    for fr in range(FH):
                        av = attr_v[b, tc, fr, pl.ds(j * 16, 16)]
                        plsc.addupdate_scatter(acc2, [iv + fr * NPAD], av)
            tn = t + NBUF

            @pl.when(tn < CHUNKS)
            def _():
                start_inputs(tn, b)
        return carry

    lax.fori_loop(0, CHUNKS // NBUF, outer, 0)

    pltpu.sync_copy(acc2, out_hbm.at[cid, sid])


ROW_BLK = 2048


def _tc_body(x_ref, p_ref, w_ref, u_ref, b_ref, o_ref):
    agg_t = jnp.sum(p_ref[...], axis=1).reshape(D_EDGE, ROW_BLK)
    w_a = w_ref[0:D_EDGE, :]
    w_x = w_ref[D_EDGE:D_EDGE + D_FEAT, :]
    w_u = w_ref[D_EDGE + D_FEAT:, :]
    o_ref[...] = (
        jnp.dot(x_ref[...], w_x, preferred_element_type=jnp.float32)
        + lax.dot_general(agg_t, w_a, (((0,), (0,)), ((), ())),
                          preferred_element_type=jnp.float32)
        + u_ref[0] * w_u
        + b_ref[...]
    )


def _tc_node_mlp(x, p, W, u, b):
    grid = (NPAD // ROW_BLK,)
    return pl.pallas_call(
        _tc_body,
        grid=grid,
        in_specs=[
            pl.BlockSpec((ROW_BLK, D_FEAT), lambda i: (i, 0)),
            pl.BlockSpec((NUM_SC, NUM_TEC, FH, ROW_BLK),
                         lambda i: (0, 0, 0, i)),
            pl.BlockSpec((D_EDGE + D_FEAT + 1, D_FEAT), lambda i: (0, 0)),
            pl.BlockSpec(memory_space=pltpu.SMEM),
            pl.BlockSpec((1, D_FEAT), lambda i: (0, 0)),
        ],
        out_specs=pl.BlockSpec((ROW_BLK, D_FEAT), lambda i: (i, 0)),
        out_shape=jax.ShapeDtypeStruct((NPAD, D_FEAT), jnp.float32),
    )(x, p, W, u, b)


def kernel(x, edge_index, edge_attr, u, W, b):
    idx2d = jnp.pad(edge_index[1].reshape(NBLK, BLK),
                    ((0, NBLK_PAD - NBLK), (0, 0)),
                    constant_values=PAD_ROW)
    attr4 = edge_attr.T.reshape(NUM_SC, FH, NBLK, BLK).transpose(0, 2, 1, 3)
    partials = _make_sc_scatter_add()(idx2d, attr4)
    partials = partials.reshape(NUM_SC, NUM_TEC, FH, NPAD)
    x_pad = jnp.pad(x, ((0, NPAD - N), (0, 0)))
    out = _tc_node_mlp(x_pad, partials, W, u.astype(jnp.float32),
                       b.reshape(1, D_FEAT))
    return out[:N]
